# Initial kernel scaffold; baseline (speedup 1.0000x reference)
import jax
import jax.numpy as jnp
from jax.experimental import pallas as pl


def kernel(x, edge_index, W1, b1, W2, b2):
    raise NotImplementedError("write your pallas kernel here")


# speedup vs baseline: 12.9738x; 12.9738x over previous
import functools

import jax
import jax.numpy as jnp
from jax import lax
from jax.experimental import pallas as pl
from jax.experimental.pallas import tpu as pltpu
from jax.experimental.pallas import tpu_sc as plsc

N = 10000
E = 320000
D = 128
NC = 2
NS = 16
NW = NC * NS
NPAD = 10240
SLICE = NPAD // NS
EPW = E // NW
C = 80
CHUNKS = EPW // C

_mesh = plsc.VectorSubcoreMesh(
    core_axis_name="c", subcore_axis_name="s", num_cores=NC, num_subcores=NS)


def _zero_vec(ref, n):
    def body(i, _):
        ref[pl.ds(i * 16, 16)] = jnp.zeros((16,), jnp.float32)
        return 0
    lax.fori_loop(0, n // 16, body, 0)


@functools.partial(
    pl.kernel,
    out_type=(jax.ShapeDtypeStruct((NC, NPAD), jnp.float32),
              jax.ShapeDtypeStruct((NC, NPAD), jnp.float32)),
    mesh=_mesh,
    scratch_types=[
        pltpu.VMEM((CHUNKS, 2, C), jnp.int32),
        pltpu.VMEM((C,), jnp.float32),
        pltpu.VMEM((SLICE,), jnp.float32),
        pltpu.VMEM_SHARED((NPAD,), jnp.float32),
        pltpu.VMEM_SHARED((NPAD,), jnp.float32),
        pltpu.SemaphoreType.DMA,
        pltpu.SemaphoreType.DMA,
    ],
)
def _deg_kernel(e_hbm, dout_hbm, din_hbm,
                idx_v, ones_v, zero_v, dout_sh, din_sh, sem_o, sem_i):
    cid = lax.axis_index("c")
    sid = lax.axis_index("s")
    w = cid * NS + sid

    pltpu.sync_copy(e_hbm.at[w], idx_v)

    def ones_body(i, _):
        ones_v[pl.ds(i * 16, 16)] = jnp.ones((16,), jnp.float32)
        return 0
    lax.fori_loop(0, C // 16, ones_body, 0)
    _zero_vec(zero_v, SLICE)
    pltpu.sync_copy(zero_v, dout_sh.at[pl.ds(sid * SLICE, SLICE)])
    pltpu.sync_copy(zero_v, din_sh.at[pl.ds(sid * SLICE, SLICE)])
    plsc.subcore_barrier()

    def fire(j, _):
        pltpu.async_copy(ones_v, dout_sh.at[idx_v.at[j, 0]], sem_o, add=True)
        pltpu.async_copy(ones_v, din_sh.at[idx_v.at[j, 1]], sem_i, add=True)
        return 0
    lax.fori_loop(0, CHUNKS, fire, 0)

    def drain(j, _):
        pltpu.make_async_copy(ones_v, dout_sh.at[idx_v.at[0, 0]], sem_o).wait()
        pltpu.make_async_copy(ones_v, din_sh.at[idx_v.at[0, 1]], sem_i).wait()
        return 0
    lax.fori_loop(0, CHUNKS, drain, 0)

    plsc.subcore_barrier()
    pltpu.sync_copy(dout_sh.at[pl.ds(sid * SLICE, SLICE)],
                    dout_hbm.at[cid, pl.ds(sid * SLICE, SLICE)])
    pltpu.sync_copy(din_sh.at[pl.ds(sid * SLICE, SLICE)],
                    din_hbm.at[cid, pl.ds(sid * SLICE, SLICE)])


def _prep_body(x_ref, w1_ref, dop_ref, dip_ref, xw_ref, ns_ref, nd_ref):
    deg_out = dop_ref[0, :] + dop_ref[1, :]
    deg_in = dip_ref[0, :] + dip_ref[1, :]
    ns = lax.rsqrt(jnp.where(deg_out > 0, deg_out, 1.0))
    nd = lax.rsqrt(jnp.where(deg_in > 0, deg_in, 1.0))
    ns_ref[:] = ns
    nd_ref[:] = nd
    xw_ref[:, :] = jnp.dot(x_ref[:, :] * ns[:N, None], w1_ref[:, :],
                           preferred_element_type=jnp.float32)


_prep = pl.pallas_call(
    _prep_body,
    out_shape=(jax.ShapeDtypeStruct((N, D), jnp.float32),
               jax.ShapeDtypeStruct((NPAD,), jnp.float32),
               jax.ShapeDtypeStruct((NPAD,), jnp.float32)),
)


@functools.partial(
    pl.kernel,
    out_type=jax.ShapeDtypeStruct((NC, NPAD), jnp.float32),
    mesh=_mesh,
    scratch_types=[
        pltpu.VMEM((CHUNKS, 2, C), jnp.int32),
        pltpu.VMEM((CHUNKS, C), jnp.float32),
        pltpu.VMEM((SLICE,), jnp.float32),
        pltpu.VMEM_SHARED((NPAD,), jnp.float32),
        pltpu.SemaphoreType.DMA,
        pltpu.SemaphoreType.DMA,
    ],
)
def _s_kernel(e_hbm, nd_hbm, s_hbm, idx_v, ndv, zero_v, s_sh, nsem, fsem):
    cid = lax.axis_index("c")
    sid = lax.axis_index("s")
    w = cid * NS + sid

    pltpu.sync_copy(e_hbm.at[w], idx_v)

    def fire_nd(j, _):
        pltpu.async_copy(nd_hbm.at[idx_v.at[j, 1]], ndv.at[j], nsem)
        return 0
    lax.fori_loop(0, CHUNKS, fire_nd, 0)

    _zero_vec(zero_v, SLICE)
    pltpu.sync_copy(zero_v, s_sh.at[pl.ds(sid * SLICE, SLICE)])
    plsc.subcore_barrier()

    def drain_nd(j, _):
        pltpu.make_async_copy(nd_hbm.at[idx_v.at[0, 1]], ndv.at[0], nsem).wait()
        return 0
    lax.fori_loop(0, CHUNKS, drain_nd, 0)

    def fire_s(j, _):
        pltpu.async_copy(ndv.at[j], s_sh.at[idx_v.at[j, 0]], fsem, add=True)
        return 0
    lax.fori_loop(0, CHUNKS, fire_s, 0)

    def drain_s(j, _):
        pltpu.make_async_copy(ndv.at[0], s_sh.at[idx_v.at[0, 0]], fsem).wait()
        return 0
    lax.fori_loop(0, CHUNKS, drain_s, 0)

    plsc.subcore_barrier()
    pltpu.sync_copy(s_sh.at[pl.ds(sid * SLICE, SLICE)],
                    s_hbm.at[cid, pl.ds(sid * SLICE, SLICE)])


@functools.partial(
    pl.kernel,
    out_type=jax.ShapeDtypeStruct((NC, NPAD, D), jnp.float32),
    mesh=_mesh,
    scratch_types=[
        pltpu.VMEM((2, 2, C), jnp.int32),
        pltpu.VMEM((2, C, D), jnp.float32),
        pltpu.VMEM_SHARED((NPAD, D), jnp.float32),
        pltpu.SemaphoreType.DMA,
        pltpu.SemaphoreType.DMA,
        pltpu.SemaphoreType.DMA,
        pltpu.SemaphoreType.DMA,
    ],
)
def _mp_kernel(xw_hbm, e_hbm, agg_hbm,
               ibuf, rows_v, agg_sh, gsa, gsb, ssa, ssb):
    cid = lax.axis_index("c")
    sid = lax.axis_index("s")
    w = cid * NS + sid

    def zr(i, _):
        for j in range(D // 16):
            rows_v[0, i, pl.ds(j * 16, 16)] = jnp.zeros((16,), jnp.float32)
        return 0
    lax.fori_loop(0, C, zr, 0)
    for r in range(SLICE // C):
        pltpu.sync_copy(rows_v.at[0], agg_sh.at[pl.ds(sid * SLICE + r * C, C)])
    plsc.subcore_barrier()

    gsem = (gsa, gsb)
    ssem = (ssa, ssb)

    def iload(j, b):
        pltpu.sync_copy(e_hbm.at[w, j], ibuf.at[b])

    def gather(j, b):
        pltpu.async_copy(xw_hbm.at[ibuf.at[b, 0]], rows_v.at[b], gsem[b])

    def gwait(b):
        pltpu.make_async_copy(xw_hbm.at[ibuf.at[b, 0]], rows_v.at[b],
                              gsem[b]).wait()

    def scatter(j, b):
        pltpu.async_copy(rows_v.at[b], agg_sh.at[ibuf.at[b, 1]], ssem[b],
                         add=True)

    def swait(b):
        pltpu.make_async_copy(rows_v.at[b], agg_sh.at[ibuf.at[b, 1]],
                              ssem[b]).wait()

    iload(0, 0)
    gather(0, 0)
    gwait(0)
    scatter(0, 0)
    iload(1, 1)
    gather(1, 1)

    def step(j, b):
        gwait(b)
        scatter(j, b)
        swait(1 - b)
        iload(j + 1, 1 - b)
        gather(j + 1, 1 - b)
        return 0

    def body_t(t, _):
        step(2 * t + 1, 1)
        step(2 * t + 2, 0)
        return 0
    lax.fori_loop(0, (CHUNKS - 3) // 2, body_t, 0)
    gwait(1)
    scatter(CHUNKS - 2, 1)
    swait(0)
    iload(CHUNKS - 1, 0)
    gather(CHUNKS - 1, 0)
    gwait(0)
    scatter(CHUNKS - 1, 0)
    swait(1)
    swait(0)

    plsc.subcore_barrier()
    pltpu.sync_copy(agg_sh.at[pl.ds(sid * SLICE, SLICE)],
                    agg_hbm.at[cid, pl.ds(sid * SLICE, SLICE)])


def _fin_body(aggp_ref, sp_ref, ns_ref, nd_ref, b1_ref, w2_ref, b2_ref,
              out_ref):
    agg = aggp_ref[0, :N, :] + aggp_ref[1, :N, :]
    h1 = jnp.maximum(agg * nd_ref[:][:N, None] + b1_ref[:][None, :], 0.0)
    c = ns_ref[:][:N] * (sp_ref[0, :N] + sp_ref[1, :N])
    r = jnp.dot(c[None, :], h1, preferred_element_type=jnp.float32)
    out_ref[:, :] = (jnp.dot(r * (1.0 / N), w2_ref[:, :],
                             preferred_element_type=jnp.float32)
                     + b2_ref[:][None, :])


_fin = pl.pallas_call(
    _fin_body,
    out_shape=jax.ShapeDtypeStruct((1, 2), jnp.float32),
)


def kernel(x, edge_index, W1, b1, W2, b2):
    edges = edge_index.reshape(2, NW, CHUNKS, C).transpose(1, 2, 0, 3)
    dout_p, din_p = _deg_kernel(edges)
    xw, ns, nd = _prep(x, W1, dout_p, din_p)
    s_p = _s_kernel(edges, nd)
    agg_p = _mp_kernel(xw, edges)
    out = _fin(agg_p, s_p, ns, nd, b1, W2, b2)
    return out.reshape(2)


# speedup vs baseline: 19.3046x; 1.4880x over previous
import functools

import jax
import jax.numpy as jnp
from jax import lax
from jax.experimental import pallas as pl
from jax.experimental.pallas import tpu as pltpu
from jax.experimental.pallas import tpu_sc as plsc

N = 10000
E = 320000
D = 128
NC = 2
NS = 16
NW = NC * NS
NPAD = 10240
SLICE = NPAD // NS
EPW = E // NW
C = 80
CHUNKS = EPW // C

_mesh = plsc.VectorSubcoreMesh(
    core_axis_name="c", subcore_axis_name="s", num_cores=NC, num_subcores=NS)


def _zero_vec(ref, n):
    def body(i, _):
        ref[pl.ds(i * 16, 16)] = jnp.zeros((16,), jnp.float32)
        return 0
    lax.fori_loop(0, n // 16, body, 0)


@functools.partial(
    pl.kernel,
    out_type=(jax.ShapeDtypeStruct((NC, NPAD), jnp.float32),
              jax.ShapeDtypeStruct((NC, NPAD), jnp.float32)),
    mesh=_mesh,
    scratch_types=[
        pltpu.VMEM((CHUNKS, 2, C), jnp.int32),
        pltpu.VMEM((C,), jnp.float32),
        pltpu.VMEM((SLICE,), jnp.float32),
        pltpu.VMEM_SHARED((NPAD,), jnp.float32),
        pltpu.VMEM_SHARED((NPAD,), jnp.float32),
        pltpu.SemaphoreType.DMA,
        pltpu.SemaphoreType.DMA,
    ],
)
def _deg_kernel(e_hbm, dout_hbm, din_hbm,
                idx_v, ones_v, zero_v, dout_sh, din_sh, sem_o, sem_i):
    cid = lax.axis_index("c")
    sid = lax.axis_index("s")
    w = cid * NS + sid

    pltpu.sync_copy(e_hbm.at[w], idx_v)

    def ones_body(i, _):
        ones_v[pl.ds(i * 16, 16)] = jnp.ones((16,), jnp.float32)
        return 0
    lax.fori_loop(0, C // 16, ones_body, 0)
    _zero_vec(zero_v, SLICE)
    pltpu.sync_copy(zero_v, dout_sh.at[pl.ds(sid * SLICE, SLICE)])
    pltpu.sync_copy(zero_v, din_sh.at[pl.ds(sid * SLICE, SLICE)])
    plsc.subcore_barrier()

    def fire(j, _):
        pltpu.async_copy(ones_v, dout_sh.at[idx_v.at[j, 0]], sem_o, add=True)
        pltpu.async_copy(ones_v, din_sh.at[idx_v.at[j, 1]], sem_i, add=True)
        return 0
    lax.fori_loop(0, CHUNKS, fire, 0)

    def drain(j, _):
        pltpu.make_async_copy(ones_v, dout_sh.at[idx_v.at[0, 0]], sem_o).wait()
        pltpu.make_async_copy(ones_v, din_sh.at[idx_v.at[0, 1]], sem_i).wait()
        return 0
    lax.fori_loop(0, CHUNKS, drain, 0)

    plsc.subcore_barrier()
    pltpu.sync_copy(dout_sh.at[pl.ds(sid * SLICE, SLICE)],
                    dout_hbm.at[cid, pl.ds(sid * SLICE, SLICE)])
    pltpu.sync_copy(din_sh.at[pl.ds(sid * SLICE, SLICE)],
                    din_hbm.at[cid, pl.ds(sid * SLICE, SLICE)])


def _prep_body(x_ref, w1_ref, dop_ref, dip_ref, xw_ref, ns_ref, nd_ref):
    deg_out = dop_ref[0, :] + dop_ref[1, :]
    deg_in = dip_ref[0, :] + dip_ref[1, :]
    ns = lax.rsqrt(jnp.where(deg_out > 0, deg_out, 1.0))
    nd = lax.rsqrt(jnp.where(deg_in > 0, deg_in, 1.0))
    ns_ref[:] = ns
    nd_ref[:] = nd
    xw_ref[:, :] = jnp.dot(x_ref[:, :] * ns[:N, None], w1_ref[:, :],
                           preferred_element_type=jnp.float32)


_prep = pl.pallas_call(
    _prep_body,
    out_shape=(jax.ShapeDtypeStruct((N, D), jnp.float32),
               jax.ShapeDtypeStruct((NPAD,), jnp.float32),
               jax.ShapeDtypeStruct((NPAD,), jnp.float32)),
)


@functools.partial(
    pl.kernel,
    out_type=(jax.ShapeDtypeStruct((NC, NPAD, D), jnp.float32),
              jax.ShapeDtypeStruct((NC, NPAD), jnp.float32)),
    mesh=_mesh,
    scratch_types=[
        pltpu.VMEM((4, 2, C), jnp.int32),
        pltpu.VMEM((2, C, D), jnp.float32),
        pltpu.VMEM((4, C), jnp.float32),
        pltpu.VMEM((SLICE,), jnp.float32),
        pltpu.VMEM_SHARED((NPAD, D), jnp.float32),
        pltpu.VMEM_SHARED((NPAD,), jnp.float32),
        pltpu.SemaphoreType.DMA,
        pltpu.SemaphoreType.DMA,
        pltpu.SemaphoreType.DMA,
        pltpu.SemaphoreType.DMA,
        pltpu.SemaphoreType.DMA,
        pltpu.SemaphoreType.DMA,
        pltpu.SemaphoreType.DMA,
        pltpu.SemaphoreType.DMA,
        pltpu.SemaphoreType.DMA,
        pltpu.SemaphoreType.DMA,
        pltpu.SemaphoreType.DMA,
        pltpu.SemaphoreType.DMA,
        pltpu.SemaphoreType.DMA,
        pltpu.SemaphoreType.DMA,
    ],
)
def _mp_kernel(xw_hbm, e_hbm, nd_hbm, agg_hbm, s_hbm,
               ibuf, rows_v, ndv, zvec_v, agg_sh, s_sh,
               gsa, gsb, ssa, ssb, is0, is1, is2, is3,
               ns0, ns1, ns2, ns3, fs0, fs1):
    cid = lax.axis_index("c")
    sid = lax.axis_index("s")
    w = cid * NS + sid

    def zr(i, _):
        for j in range(D // 16):
            rows_v[0, i, pl.ds(j * 16, 16)] = jnp.zeros((16,), jnp.float32)
        return 0
    lax.fori_loop(0, C, zr, 0)
    for r in range(SLICE // C):
        pltpu.sync_copy(rows_v.at[0], agg_sh.at[pl.ds(sid * SLICE + r * C, C)])
    _zero_vec(zvec_v, SLICE)
    pltpu.sync_copy(zvec_v, s_sh.at[pl.ds(sid * SLICE, SLICE)])
    plsc.subcore_barrier()

    gsem = (gsa, gsb)
    ssem = (ssa, ssb)
    isem = (is0, is1, is2, is3)
    nsem = (ns0, ns1, ns2, ns3)
    fsem = (fs0, fs1)

    def iload(j, s4):
        pltpu.async_copy(e_hbm.at[w, j], ibuf.at[s4], isem[s4])

    def iwait(s4):
        pltpu.make_async_copy(e_hbm.at[w, 0], ibuf.at[s4], isem[s4]).wait()

    def gather(s4, b):
        pltpu.async_copy(xw_hbm.at[ibuf.at[s4, 0]], rows_v.at[b], gsem[b])

    def gwait(b):
        pltpu.make_async_copy(xw_hbm.at[ibuf.at[0, 0]], rows_v.at[b],
                              gsem[b]).wait()

    def scatter(s4, b):
        pltpu.async_copy(rows_v.at[b], agg_sh.at[ibuf.at[s4, 1]], ssem[b],
                         add=True)

    def swait(b):
        pltpu.make_async_copy(rows_v.at[b], agg_sh.at[ibuf.at[0, 1]],
                              ssem[b]).wait()

    def ndfire(s4):
        pltpu.async_copy(nd_hbm.at[ibuf.at[s4, 1]], ndv.at[s4], nsem[s4])

    def ndwait(s4):
        pltpu.make_async_copy(nd_hbm.at[ibuf.at[0, 1]], ndv.at[s4],
                              nsem[s4]).wait()

    def sfire(s4, b):
        pltpu.async_copy(ndv.at[s4], s_sh.at[ibuf.at[s4, 0]], fsem[b],
                         add=True)

    def sdone(b):
        pltpu.make_async_copy(ndv.at[0], s_sh.at[ibuf.at[0, 0]],
                              fsem[b]).wait()

    iload(0, 0)
    iload(1, 1)
    iload(2, 2)
    iwait(0)
    gather(0, 0)
    ndfire(0)
    gwait(0)
    scatter(0, 0)
    ndwait(0)
    sfire(0, 0)
    iload(3, 3)
    iwait(1)
    gather(1, 1)
    ndfire(1)

    def step(j, s4, b):
        gwait(b)
        scatter(s4, b)
        ndwait(s4)
        sfire(s4, b)
        swait(1 - b)
        sdone(1 - b)
        iload(j + 3, (s4 + 3) % 4)
        iwait((s4 + 1) % 4)
        gather((s4 + 1) % 4, 1 - b)
        ndfire((s4 + 1) % 4)
        return 0

    def body_t(t, _):
        j = 4 * t + 1
        step(j, 1, 1)
        step(j + 1, 2, 0)
        step(j + 2, 3, 1)
        step(j + 3, 0, 0)
        return 0
    lax.fori_loop(0, (CHUNKS - 5) // 4, body_t, 0)
    gwait(1)
    scatter(1, 1)
    ndwait(1)
    sfire(1, 1)
    swait(0)
    sdone(0)
    iload(CHUNKS - 1, 0)
    iwait(2)
    gather(2, 0)
    ndfire(2)
    gwait(0)
    scatter(2, 0)
    ndwait(2)
    sfire(2, 0)
    swait(1)
    sdone(1)
    iwait(3)
    gather(3, 1)
    ndfire(3)
    gwait(1)
    scatter(3, 1)
    ndwait(3)
    sfire(3, 1)
    swait(0)
    sdone(0)
    iwait(0)
    gather(0, 0)
    ndfire(0)
    gwait(0)
    scatter(0, 0)
    ndwait(0)
    sfire(0, 0)
    swait(1)
    sdone(1)
    swait(0)
    sdone(0)

    plsc.subcore_barrier()
    pltpu.sync_copy(agg_sh.at[pl.ds(sid * SLICE, SLICE)],
                    agg_hbm.at[cid, pl.ds(sid * SLICE, SLICE)])
    pltpu.sync_copy(s_sh.at[pl.ds(sid * SLICE, SLICE)],
                    s_hbm.at[cid, pl.ds(sid * SLICE, SLICE)])


def _fin_body(aggp_ref, sp_ref, ns_ref, nd_ref, b1_ref, w2_ref, b2_ref,
              out_ref):
    agg = aggp_ref[0, :N, :] + aggp_ref[1, :N, :]
    h1 = jnp.maximum(agg * nd_ref[:][:N, None] + b1_ref[:][None, :], 0.0)
    c = ns_ref[:][:N] * (sp_ref[0, :N] + sp_ref[1, :N])
    r = jnp.dot(c[None, :], h1, preferred_element_type=jnp.float32)
    out_ref[:, :] = (jnp.dot(r * (1.0 / N), w2_ref[:, :],
                             preferred_element_type=jnp.float32)
                     + b2_ref[:][None, :])


_fin = pl.pallas_call(
    _fin_body,
    out_shape=jax.ShapeDtypeStruct((1, 2), jnp.float32),
)


def kernel(x, edge_index, W1, b1, W2, b2):
    edges = edge_index.reshape(2, NW, CHUNKS, C).transpose(1, 2, 0, 3)
    dout_p, din_p = _deg_kernel(edges)
    xw, ns, nd = _prep(x, W1, dout_p, din_p)
    agg_p, s_p = _mp_kernel(xw, edges, nd)
    out = _fin(agg_p, s_p, ns, nd, b1, W2, b2)
    return out.reshape(2)
